# Initial kernel scaffold; baseline (speedup 1.0000x reference)
import jax
import jax.numpy as jnp
from jax.experimental import pallas as pl


def kernel(values, table):
    raise NotImplementedError("write your pallas kernel here")


# speedup vs baseline: 189.5231x; 189.5231x over previous
import functools

import jax
import jax.numpy as jnp
from jax import lax
from jax.experimental import pallas as pl
from jax.experimental.pallas import tpu as pltpu
from jax.experimental.pallas import tpu_sc as plsc

_N = 16384 * 200
_NC = 2
_NS = 16
_NW = _NC * _NS
_PER_W = _N // _NW
_CHUNK = 6400
_NCHUNK = _PER_W // _CHUNK
_LANES = 16
_TPAD = 96

_mesh = plsc.VectorSubcoreMesh(core_axis_name="c", subcore_axis_name="s")


@functools.partial(
    pl.kernel,
    mesh=_mesh,
    out_type=jax.ShapeDtypeStruct((_N,), jnp.int32),
    compiler_params=pltpu.CompilerParams(needs_layout_passes=False),
    scratch_types=[
        pltpu.VMEM((_TPAD,), jnp.int32),
        pltpu.VMEM((_CHUNK,), jnp.int32),
        pltpu.VMEM((_CHUNK,), jnp.int32),
    ],
)
def _lookup(values_hbm, table_hbm, out_hbm, table_v, in_v, out_v):
    wid = lax.axis_index("s") * _NC + lax.axis_index("c")
    base = wid * _PER_W
    pltpu.sync_copy(table_hbm, table_v)

    def chunk_body(g, carry):
        off = base + g * _CHUNK
        pltpu.sync_copy(values_hbm.at[pl.ds(off, _CHUNK)], in_v)

        def gather_body(i, c):
            idx = in_v[pl.ds(i * _LANES, _LANES)]
            out_v[pl.ds(i * _LANES, _LANES)] = plsc.load_gather(table_v, [idx])
            return c

        lax.fori_loop(0, _CHUNK // _LANES, gather_body, 0, unroll=8)
        pltpu.sync_copy(out_v, out_hbm.at[pl.ds(off, _CHUNK)])
        return carry

    lax.fori_loop(0, _NCHUNK, chunk_body, 0)


def kernel(values, table):
    table_p = jnp.pad(table, (0, _TPAD - table.shape[0]))
    out = _lookup(values.reshape(_N), table_p)
    return out.reshape(values.shape)


# speedup vs baseline: 202.8274x; 1.0702x over previous
import functools

import jax
import jax.numpy as jnp
from jax import lax
from jax.experimental import pallas as pl
from jax.experimental.pallas import tpu as pltpu
from jax.experimental.pallas import tpu_sc as plsc

_N = 16384 * 200
_NC = 2
_NS = 16
_NW = _NC * _NS
_PER_W = _N // _NW
_CHUNK = 12800
_NCHUNK = _PER_W // _CHUNK
_NB = 2
_LANES = 16
_TPAD = 96

_mesh = plsc.VectorSubcoreMesh(core_axis_name="c", subcore_axis_name="s")


@functools.partial(
    pl.kernel,
    mesh=_mesh,
    out_type=jax.ShapeDtypeStruct((_N,), jnp.int32),
    compiler_params=pltpu.CompilerParams(needs_layout_passes=False),
    scratch_types=[
        pltpu.VMEM((_TPAD,), jnp.int32),
        pltpu.VMEM((_NB, _CHUNK), jnp.int32),
        pltpu.VMEM((_NB, _CHUNK), jnp.int32),
        pltpu.SemaphoreType.DMA((_NB,)),
        pltpu.SemaphoreType.DMA((_NB,)),
    ],
)
def _lookup(values_hbm, table_hbm, out_hbm, table_v, in_v, out_v, insem, outsem):
    wid = lax.axis_index("s") * _NC + lax.axis_index("c")
    base = wid * _PER_W
    pltpu.sync_copy(table_hbm, table_v)

    def gather_chunk(b):
        @pl.loop(0, _CHUNK // _LANES, unroll=8)
        def _(i):
            idx = in_v[b, pl.ds(i * _LANES, _LANES)]
            out_v[b, pl.ds(i * _LANES, _LANES)] = plsc.load_gather(
                table_v, [idx])

    for b in range(_NB):
        pltpu.async_copy(
            values_hbm.at[pl.ds(base + b * _CHUNK, _CHUNK)],
            in_v.at[b], insem.at[b])

    for b in range(_NB):
        off = base + b * _CHUNK
        pltpu.make_async_copy(
            values_hbm.at[pl.ds(off, _CHUNK)], in_v.at[b], insem.at[b]).wait()
        gather_chunk(b)
        pltpu.async_copy(out_v.at[b], out_hbm.at[pl.ds(off, _CHUNK)],
                         outsem.at[b])
        nxt = b + _NB
        if nxt < _NCHUNK:
            pltpu.async_copy(
                values_hbm.at[pl.ds(base + nxt * _CHUNK, _CHUNK)],
                in_v.at[b], insem.at[b])

    @pl.loop(_NB, _NCHUNK, step=_NB)
    def _(g0):
        for b in range(_NB):
            off = base + (g0 + b) * _CHUNK
            pltpu.make_async_copy(
                values_hbm.at[pl.ds(off, _CHUNK)], in_v.at[b],
                insem.at[b]).wait()
            pltpu.make_async_copy(
                out_v.at[b], out_hbm.at[pl.ds(off, _CHUNK)],
                outsem.at[b]).wait()
            gather_chunk(b)
            pltpu.async_copy(out_v.at[b], out_hbm.at[pl.ds(off, _CHUNK)],
                             outsem.at[b])

            @pl.when(g0 + b + _NB < _NCHUNK)
            def _():
                pltpu.async_copy(
                    values_hbm.at[pl.ds(base + (g0 + b + _NB) * _CHUNK,
                                        _CHUNK)],
                    in_v.at[b], insem.at[b])

    for b in range(_NB):
        off = base + (_NCHUNK - _NB + b) * _CHUNK
        pltpu.make_async_copy(
            out_v.at[b], out_hbm.at[pl.ds(off, _CHUNK)], outsem.at[b]).wait()


def kernel(values, table):
    table_p = jnp.pad(table, (0, _TPAD - table.shape[0]))
    out = _lookup(values.reshape(_N), table_p)
    return out.reshape(values.shape)


# speedup vs baseline: 518.1961x; 2.5549x over previous
import functools

import jax
import jax.numpy as jnp
from jax import lax
from jax.experimental import pallas as pl
from jax.experimental.pallas import tpu as pltpu
from jax.experimental.pallas import tpu_sc as plsc

_ROWS = 16384
_COLS = 200
_NC = 2
_NS = 16
_NW = _NC * _NS
_ROWS_W = _ROWS // _NW
_RCHUNK = 64
_NCHUNK = _ROWS_W // _RCHUNK
_NB = 2
_LANES = 16
_TPAD = 96
_VPP = (2 * _COLS) // _LANES

_mesh = plsc.VectorSubcoreMesh(core_axis_name="c", subcore_axis_name="s")


@functools.partial(
    pl.kernel,
    mesh=_mesh,
    out_type=jax.ShapeDtypeStruct((_ROWS, _COLS), jnp.int32),
    compiler_params=pltpu.CompilerParams(needs_layout_passes=False),
    scratch_types=[
        pltpu.VMEM((_TPAD,), jnp.int32),
        pltpu.VMEM((_NB, _RCHUNK, _COLS), jnp.int32),
        pltpu.VMEM((_NB, _RCHUNK, _COLS), jnp.int32),
        pltpu.SemaphoreType.DMA((_NB,)),
        pltpu.SemaphoreType.DMA((_NB,)),
    ],
)
def _lookup(values_hbm, table_hbm, out_hbm, table_v, in_v, out_v, insem,
            outsem):
    wid = lax.axis_index("s") * _NC + lax.axis_index("c")
    base = wid * _ROWS_W
    pltpu.sync_copy(table_hbm, table_v)

    iota = lax.iota(jnp.int32, _LANES)
    in_hi = iota >= 8
    bnd_row = jnp.where(in_hi, 1, 0)
    bnd_col = jnp.where(in_hi, iota - 8, iota + 192)

    def gather_chunk(b):
        ib = in_v.at[b]
        ob = out_v.at[b]

        @plsc.parallel_loop(0, _RCHUNK // 2, unroll=1)
        def _(q):
            r = 2 * q
            for j in range(_VPP):
                if j == 12:
                    rows = bnd_row + r
                    idx = plsc.load_gather(ib, [rows, bnd_col])
                    res = plsc.load_gather(table_v, [idx])
                    plsc.store_scatter(ob, [rows, bnd_col], res)
                else:
                    rr = r if j < 12 else r + 1
                    c0 = 16 * j if j < 12 else 16 * j - 200
                    idx = ib[rr, pl.ds(c0, _LANES)]
                    ob[rr, pl.ds(c0, _LANES)] = plsc.load_gather(
                        table_v, [idx])

    for b in range(_NB):
        pltpu.async_copy(
            values_hbm.at[pl.ds(base + b * _RCHUNK, _RCHUNK), :],
            in_v.at[b], insem.at[b])

    for b in range(_NB):
        r0 = base + b * _RCHUNK
        pltpu.make_async_copy(
            values_hbm.at[pl.ds(r0, _RCHUNK), :], in_v.at[b],
            insem.at[b]).wait()
        gather_chunk(b)
        pltpu.async_copy(out_v.at[b], out_hbm.at[pl.ds(r0, _RCHUNK), :],
                         outsem.at[b])
        nxt = b + _NB
        if nxt < _NCHUNK:
            pltpu.async_copy(
                values_hbm.at[pl.ds(base + nxt * _RCHUNK, _RCHUNK), :],
                in_v.at[b], insem.at[b])

    @pl.loop(_NB, _NCHUNK, step=_NB)
    def _(g0):
        for b in range(_NB):
            r0 = base + (g0 + b) * _RCHUNK
            pltpu.make_async_copy(
                values_hbm.at[pl.ds(r0, _RCHUNK), :], in_v.at[b],
                insem.at[b]).wait()
            pltpu.make_async_copy(
                out_v.at[b], out_hbm.at[pl.ds(r0, _RCHUNK), :],
                outsem.at[b]).wait()
            gather_chunk(b)
            pltpu.async_copy(out_v.at[b], out_hbm.at[pl.ds(r0, _RCHUNK), :],
                             outsem.at[b])

            @pl.when(g0 + b + _NB < _NCHUNK)
            def _():
                pltpu.async_copy(
                    values_hbm.at[pl.ds(base + (g0 + b + _NB) * _RCHUNK,
                                        _RCHUNK), :],
                    in_v.at[b], insem.at[b])

    for b in range(_NB):
        r0 = base + (_NCHUNK - _NB + b) * _RCHUNK
        pltpu.make_async_copy(
            out_v.at[b], out_hbm.at[pl.ds(r0, _RCHUNK), :],
            outsem.at[b]).wait()


def kernel(values, table):
    table_p = jnp.pad(table, (0, _TPAD - table.shape[0]))
    return _lookup(values, table_p)


# speedup vs baseline: 519.9540x; 1.0034x over previous
import functools

import jax
import jax.numpy as jnp
from jax import lax
from jax.experimental import pallas as pl
from jax.experimental.pallas import tpu as pltpu
from jax.experimental.pallas import tpu_sc as plsc

_ROWS = 16384
_COLS = 200
_NC = 2
_NS = 16
_NW = _NC * _NS
_ROWS_W = _ROWS // _NW
_RCHUNK = 64
_NCHUNK = _ROWS_W // _RCHUNK
_NB = 2
_LANES = 16
_TPAD = 96
_C0S = tuple(range(0, _COLS - _LANES, _LANES)) + (_COLS - _LANES,)

_mesh = plsc.VectorSubcoreMesh(core_axis_name="c", subcore_axis_name="s")


def _make(interpret=False):
    return functools.partial(
        pl.kernel,
        mesh=_mesh,
        out_type=jax.ShapeDtypeStruct((_ROWS, _COLS), jnp.int32),
        compiler_params=pltpu.CompilerParams(needs_layout_passes=False),
        scratch_types=[
            pltpu.VMEM((_TPAD,), jnp.int32),
            pltpu.VMEM((_NB, _RCHUNK, _COLS), jnp.int32),
            pltpu.VMEM((_NB, _RCHUNK, _COLS), jnp.int32),
            pltpu.SemaphoreType.DMA((_NB,)),
            pltpu.SemaphoreType.DMA((_NB,)),
        ],
        interpret=interpret,
    )


def _lookup_body(values_hbm, table_hbm, out_hbm, table_v, in_v, out_v, insem,
                 outsem):
    wid = lax.axis_index("s") * _NC + lax.axis_index("c")
    base = wid * _ROWS_W
    pltpu.sync_copy(table_hbm, table_v)

    def gather_chunk(b):
        ib = in_v.at[b]
        ob = out_v.at[b]

        @plsc.parallel_loop(0, _RCHUNK, unroll=2)
        def _(r):
            for c0 in _C0S:
                idx = ib[r, pl.ds(c0, _LANES)]
                ob[r, pl.ds(c0, _LANES)] = plsc.load_gather(table_v, [idx])

    for b in range(_NB):
        pltpu.async_copy(
            values_hbm.at[pl.ds(base + b * _RCHUNK, _RCHUNK), :],
            in_v.at[b], insem.at[b])

    for b in range(_NB):
        r0 = base + b * _RCHUNK
        pltpu.make_async_copy(
            values_hbm.at[pl.ds(r0, _RCHUNK), :], in_v.at[b],
            insem.at[b]).wait()
        gather_chunk(b)
        pltpu.async_copy(out_v.at[b], out_hbm.at[pl.ds(r0, _RCHUNK), :],
                         outsem.at[b])
        nxt = b + _NB
        if nxt < _NCHUNK:
            pltpu.async_copy(
                values_hbm.at[pl.ds(base + nxt * _RCHUNK, _RCHUNK), :],
                in_v.at[b], insem.at[b])

    @pl.loop(_NB, _NCHUNK, step=_NB)
    def _(g0):
        for b in range(_NB):
            r0 = base + (g0 + b) * _RCHUNK
            pltpu.make_async_copy(
                values_hbm.at[pl.ds(r0, _RCHUNK), :], in_v.at[b],
                insem.at[b]).wait()
            pltpu.make_async_copy(
                out_v.at[b], out_hbm.at[pl.ds(r0, _RCHUNK), :],
                outsem.at[b]).wait()
            gather_chunk(b)
            pltpu.async_copy(out_v.at[b], out_hbm.at[pl.ds(r0, _RCHUNK), :],
                             outsem.at[b])

            @pl.when(g0 + b + _NB < _NCHUNK)
            def _():
                pltpu.async_copy(
                    values_hbm.at[pl.ds(base + (g0 + b + _NB) * _RCHUNK,
                                        _RCHUNK), :],
                    in_v.at[b], insem.at[b])

    for b in range(_NB):
        r0 = base + (_NCHUNK - _NB + b) * _RCHUNK
        pltpu.make_async_copy(
            out_v.at[b], out_hbm.at[pl.ds(r0, _RCHUNK), :],
            outsem.at[b]).wait()


_lookup = _make()(_lookup_body)


def kernel(values, table):
    table_p = jnp.pad(table, (0, _TPAD - table.shape[0]))
    return _lookup(values, table_p)


# speedup vs baseline: 961.5547x; 1.8493x over previous
import functools

import jax
import jax.numpy as jnp
from jax import lax
from jax.experimental import pallas as pl
from jax.experimental.pallas import tpu as pltpu
from jax.experimental.pallas import tpu_sc as plsc

_ROWS = 200
_COLS = 16384
_NC = 2
_NS = 16
_NW = _NC * _NS
_COLS_W = _COLS // _NW
_CCHUNK = 128
_NCHUNK = _COLS_W // _CCHUNK
_NB = 2
_LANES = 16
_TPAD = 96

_mesh = plsc.VectorSubcoreMesh(core_axis_name="c", subcore_axis_name="s")


def _make(interpret=False):
    return functools.partial(
        pl.kernel,
        mesh=_mesh,
        out_type=jax.ShapeDtypeStruct((_ROWS, _COLS), jnp.int32),
        compiler_params=pltpu.CompilerParams(needs_layout_passes=False),
        scratch_types=[
            pltpu.VMEM((_TPAD,), jnp.int32),
            pltpu.VMEM((_NB, _ROWS, _CCHUNK), jnp.int32),
            pltpu.VMEM((_NB, _ROWS, _CCHUNK), jnp.int32),
            pltpu.SemaphoreType.DMA((_NB,)),
            pltpu.SemaphoreType.DMA((_NB,)),
        ],
        interpret=interpret,
    )


def _lookup_body(values_hbm, table_hbm, out_hbm, table_v, in_v, out_v, insem,
                 outsem):
    wid = lax.axis_index("s") * _NC + lax.axis_index("c")
    c0w = wid * _COLS_W
    pltpu.sync_copy(table_hbm, table_v)

    def gather_chunk(b):
        ib = in_v.at[b]
        ob = out_v.at[b]

        @plsc.parallel_loop(0, _ROWS, unroll=1)
        def _(r):
            for c0 in range(0, _CCHUNK, _LANES):
                idx = ib[r, pl.ds(c0, _LANES)]
                ob[r, pl.ds(c0, _LANES)] = plsc.load_gather(table_v, [idx])

    def in_slice(g):
        return values_hbm.at[pl.ds(0, _ROWS),
                             pl.ds(c0w + g * _CCHUNK, _CCHUNK)]

    def out_slice(g):
        return out_hbm.at[pl.ds(0, _ROWS), pl.ds(c0w + g * _CCHUNK, _CCHUNK)]

    for b in range(_NB):
        pltpu.async_copy(in_slice(b), in_v.at[b], insem.at[b])

    for b in range(_NB):
        pltpu.make_async_copy(in_slice(b), in_v.at[b], insem.at[b]).wait()
        gather_chunk(b)
        pltpu.async_copy(out_v.at[b], out_slice(b), outsem.at[b])
        if b + _NB < _NCHUNK:
            pltpu.async_copy(in_slice(b + _NB), in_v.at[b], insem.at[b])

    @pl.loop(_NB, _NCHUNK, step=_NB)
    def _(g0):
        for b in range(_NB):
            g = g0 + b
            pltpu.make_async_copy(in_slice(g), in_v.at[b], insem.at[b]).wait()
            pltpu.make_async_copy(out_v.at[b], out_slice(g),
                                  outsem.at[b]).wait()
            gather_chunk(b)
            pltpu.async_copy(out_v.at[b], out_slice(g), outsem.at[b])

            @pl.when(g + _NB < _NCHUNK)
            def _():
                pltpu.async_copy(in_slice(g + _NB), in_v.at[b], insem.at[b])

    for b in range(_NB):
        pltpu.make_async_copy(out_v.at[b], out_slice(_NCHUNK - _NB + b),
                              outsem.at[b]).wait()


_lookup = _make()(_lookup_body)


def kernel(values, table):
    table_p = jnp.pad(table, (0, _TPAD - table.shape[0]))
    out_t = _lookup(jnp.swapaxes(values, 0, 1), table_p)
    return jnp.swapaxes(out_t, 0, 1)
